# baseline (speedup 1.0000x reference)
import functools

import jax
import jax.numpy as jnp
from jax import lax
from jax.experimental import pallas as pl
from jax.experimental.pallas import tpu as pltpu
from jax.experimental.pallas import tpu_sc as plsc

VOCAB = 1000000
EMBED = 64
BATCH = 16384
CTX = 5

NC = 2
NS = 16
L = 16
NW = NC * NS
BPW = BATCH // NW
CHUNK = 128
NCHUNK = BPW // CHUNK
EV = EMBED // L


def _body(tgt_hbm, ctx_hbm, ttab_hbm, ctab_hbm, out_hbm,
          idx_t, idx_c, wbuf, cbuf, obuf, sem):
    wid = lax.axis_index("s") * NC + lax.axis_index("c")
    base = wid * BPW

    lane = lax.iota(jnp.int32, L)
    last_lane = lane == (L - 1)

    for k in range(NCHUNK):
        cb = base + k * CHUNK
        pltpu.sync_copy(tgt_hbm.at[pl.ds(cb, CHUNK)], idx_t)
        pltpu.sync_copy(ctx_hbm.at[pl.ds(cb * CTX, CHUNK * CTX)], idx_c)
        copies = [pltpu.async_copy(ttab_hbm.at[idx_t], wbuf, sem)]
        for g in range(CTX):
            copies.append(pltpu.async_copy(
                ctab_hbm.at[idx_c.at[pl.ds(g * CHUNK, CHUNK)]],
                cbuf.at[pl.ds(g * CHUNK, CHUNK)], sem))
        zero = jnp.zeros((L,), jnp.float32)
        for i in range(CHUNK * CTX // L):
            obuf[pl.ds(i * L, L)] = zero
        for c in copies:
            c.wait()

        def b_body(b, carry):
            w = [wbuf[b, pl.ds(j * L, L)] for j in range(EV)]
            for c in range(CTX):
                r = b * CTX + c
                acc = w[0] * cbuf[r, pl.ds(0, L)]
                for j in range(1, EV):
                    acc = acc + w[j] * cbuf[r, pl.ds(j * L, L)]
                plsc.addupdate_scatter(obuf, [jnp.full((L,), r, jnp.int32)],
                                       acc)
            return carry

        lax.fori_loop(0, CHUNK, b_body, 0)
        pltpu.sync_copy(obuf, out_hbm.at[pl.ds(cb * CTX, CHUNK * CTX)])


@jax.jit
def kernel(target, context, target_table, context_table):
    mesh = plsc.VectorSubcoreMesh(core_axis_name="c", subcore_axis_name="s",
                                  num_cores=NC, num_subcores=NS)
    ctx_flat = context.reshape(BATCH * CTX)
    run = functools.partial(
        pl.kernel,
        out_type=jax.ShapeDtypeStruct((BATCH * CTX,), jnp.float32),
        mesh=mesh,
        scratch_types=[
            pltpu.VMEM((CHUNK,), jnp.int32),
            pltpu.VMEM((CHUNK * CTX,), jnp.int32),
            pltpu.VMEM((CHUNK, EMBED), jnp.float32),
            pltpu.VMEM((CHUNK * CTX, EMBED), jnp.float32),
            pltpu.VMEM((CHUNK * CTX,), jnp.float32),
            pltpu.SemaphoreType.DMA,
        ],
        compiler_params=pltpu.CompilerParams(needs_layout_passes=False,
                                             use_tc_tiling_on_sc=False),
    )(_body)
    out = run(target, ctx_flat, target_table, context_table)
    return out.reshape(BATCH, CTX)
